# baseline (speedup 1.0000x reference)
import jax
import jax.numpy as jnp
from jax.experimental import pallas as pl
from jax.experimental.pallas import tpu as pltpu

N = 8192
D = 512
DA = D + 16
BR = 256
CH = 512
NCH = N // CH
NI = N // BR
_SQRT2 = 1.4142135623730951


def _emb_loss_kernel(w_hbm, out_ref, lhs_ref, rhs_ref, m_scratch, sq_ref,
                     wtmp0, wtmp1, sem0, sem1):
    bufs = (wtmp0, wtmp1)
    sems = (sem0, sem1)
    ones_d = jnp.ones((D, 1), jnp.float32)

    def chunk_copy(k):
        return pltpu.make_async_copy(
            w_hbm.at[pl.ds(k * CH, CH), :], bufs[k % 2], sems[k % 2])

    chunk_copy(0).start()
    for k in range(NCH):
        if k + 1 < NCH:
            chunk_copy(k + 1).start()
        chunk_copy(k).wait()
        blk = bufs[k % 2][...]
        sq_ref[k * CH:(k + 1) * CH, :] = jax.lax.dot_general(
            blk * blk, ones_d,
            dimension_numbers=(((1,), (0,)), ((), ())),
            preferred_element_type=jnp.float32,
        )
        lhs_ref[k * CH:(k + 1) * CH, :D] = (blk * -_SQRT2).astype(jnp.bfloat16)
        rhs_ref[k * CH:(k + 1) * CH, :D] = (blk * _SQRT2).astype(jnp.bfloat16)

    sq_col = sq_ref[...]
    sq_hi = sq_col.astype(jnp.bfloat16)
    sq_lo = (sq_col - sq_hi.astype(jnp.float32)).astype(jnp.bfloat16)
    lhs_ref[:, D:] = jnp.concatenate(
        [jnp.ones((N, 2), jnp.bfloat16), jnp.zeros((N, 14), jnp.bfloat16)],
        axis=1)
    rhs_ref[:, D:] = jnp.concatenate(
        [sq_hi, sq_lo, jnp.zeros((N, 14), jnp.bfloat16)], axis=1)

    r = jax.lax.broadcasted_iota(jnp.int32, (BR, BR), 0)
    c = jax.lax.broadcasted_iota(jnp.int32, (BR, BR), 1)
    eye_inf = jnp.where(r == c, jnp.inf, 0.0).astype(jnp.float32)

    rhs = rhs_ref[...]
    for i in range(NI):
        t = jax.lax.dot_general(
            lhs_ref[i * BR:(i + 1) * BR, :], rhs,
            dimension_numbers=(((1,), (1,)), ((), ())),
            preferred_element_type=jnp.float32,
        )
        lo, hi = i * BR, (i + 1) * BR
        m = jnp.min(t[:, lo:hi] + eye_inf, axis=1, keepdims=True)
        if lo > 0:
            m = jnp.minimum(m, jnp.min(t[:, :lo], axis=1, keepdims=True))
        if hi < N:
            m = jnp.minimum(m, jnp.min(t[:, hi:], axis=1, keepdims=True))
        m_scratch[lo:hi, :] = m

    min_d2 = sq_col + m_scratch[...]
    d = jnp.sqrt(jnp.maximum(min_d2, 1e-12))
    mean = jnp.sum(d) / N
    kept = jnp.where(d > mean, 0.0, d)
    loss = -(jnp.sum(kept) / N)
    out_ref[0] = loss
    out_ref[1] = mean


def kernel(weight):
    out = pl.pallas_call(
        _emb_loss_kernel,
        in_specs=[pl.BlockSpec(memory_space=pl.ANY)],
        out_specs=pl.BlockSpec(memory_space=pltpu.SMEM),
        out_shape=jax.ShapeDtypeStruct((2,), jnp.float32),
        scratch_shapes=[
            pltpu.VMEM((N, DA), jnp.bfloat16),
            pltpu.VMEM((N, DA), jnp.bfloat16),
            pltpu.VMEM((N, 1), jnp.float32),
            pltpu.VMEM((N, 1), jnp.float32),
            pltpu.VMEM((CH, D), jnp.float32),
            pltpu.VMEM((CH, D), jnp.float32),
            pltpu.SemaphoreType.DMA,
            pltpu.SemaphoreType.DMA,
        ],
    )(weight)
    return out


# speedup vs baseline: 2.5578x; 2.5578x over previous
import jax
import jax.numpy as jnp
from jax.experimental import pallas as pl
from jax.experimental.pallas import tpu as pltpu

N = 8192
D = 512
BC = 512
NB = N // BC
RC = 2048
_SQRT2 = 1.4142135623730951


def _emb_loss_kernel(w_ref, out_ref, rowacc, colacc):
    w_full = w_ref[...]
    w2 = w_full * w_full

    sq_row = jax.lax.dot_general(
        jnp.ones((1, D), jnp.float32), w2,
        dimension_numbers=(((1,), (1,)), ((), ())),
        preferred_element_type=jnp.float32,
    )
    sq_col = jax.lax.dot_general(
        w2, jnp.ones((D, 1), jnp.float32),
        dimension_numbers=(((1,), (0,)), ((), ())),
        preferred_element_type=jnp.float32,
    )

    wsbf = (w_full * _SQRT2).astype(jnp.bfloat16)

    rowacc[...] = jnp.full((N, 1), jnp.inf, jnp.float32)

    r = jax.lax.broadcasted_iota(jnp.int32, (BC, BC), 0)
    c = jax.lax.broadcasted_iota(jnp.int32, (BC, BC), 1)
    eye_inf = jnp.where(r == c, jnp.inf, 0.0).astype(jnp.float32)

    for j in range(NB):
        cl, chi = j * BC, (j + 1) * BC
        rhs = wsbf[cl:chi, :]
        colmin = None
        base = 0
        while base < chi:
            sz = min(RC, chi - base)
            g = jax.lax.dot_general(
                wsbf[base:base + sz, :], rhs,
                dimension_numbers=(((1,), (1,)), ((), ())),
                preferred_element_type=jnp.float32,
            )
            if base <= cl < base + sz:
                loc = cl - base
                pieces = []
                if loc > 0:
                    pieces.append(g[:loc, :])
                pieces.append(g[loc:loc + BC, :] - eye_inf)
                if loc + BC < sz:
                    pieces.append(g[loc + BC:, :])
                g = (jnp.concatenate(pieces, axis=0)
                     if len(pieces) > 1 else pieces[0])
            t1 = sq_col[base:base + sz, :] - g
            part_col = jnp.min(t1, axis=0, keepdims=True)
            colmin = (part_col if colmin is None
                      else jnp.minimum(colmin, part_col))
            t2 = sq_row[:, cl:chi] - g
            part_row = jnp.min(t2, axis=1, keepdims=True)
            rowacc[base:base + sz, :] = jnp.minimum(
                rowacc[base:base + sz, :], part_row)
            base += sz
        colacc[:, cl:chi] = colmin

    m = jnp.minimum(rowacc[...], jnp.reshape(colacc[...], (N, 1)))
    min_d2 = sq_col + m
    d = jnp.sqrt(jnp.maximum(min_d2, 1e-12))
    mean = jnp.sum(d) / N
    kept = jnp.where(d > mean, 0.0, d)
    loss = -(jnp.sum(kept) / N)
    out_ref[0] = loss
    out_ref[1] = mean


def kernel(weight):
    out = pl.pallas_call(
        _emb_loss_kernel,
        in_specs=[pl.BlockSpec((N, D), lambda: (0, 0))],
        out_specs=pl.BlockSpec(memory_space=pltpu.SMEM),
        out_shape=jax.ShapeDtypeStruct((2,), jnp.float32),
        scratch_shapes=[
            pltpu.VMEM((N, 1), jnp.float32),
            pltpu.VMEM((1, N), jnp.float32),
        ],
    )(weight)
    return out


# speedup vs baseline: 2.6205x; 1.0245x over previous
import jax
import jax.numpy as jnp
from jax.experimental import pallas as pl
from jax.experimental.pallas import tpu as pltpu

N = 8192
D = 512
BC = 512
NB = N // BC
RC = 2048
_SQRT2 = 1.4142135623730951


def _emb_loss_kernel(w_ref, out_ref, rowacc, colacc):
    w_full = w_ref[...]
    w2 = w_full * w_full

    sq_row = jax.lax.dot_general(
        jnp.ones((1, D), jnp.float32), w2,
        dimension_numbers=(((1,), (1,)), ((), ())),
        preferred_element_type=jnp.float32,
    )
    sq_col = jax.lax.dot_general(
        w2, jnp.ones((D, 1), jnp.float32),
        dimension_numbers=(((1,), (0,)), ((), ())),
        preferred_element_type=jnp.float32,
    )

    wsbf = (w_full * _SQRT2).astype(jnp.bfloat16)

    rowacc[...] = jnp.full((N, 128), jnp.inf, jnp.float32)

    r = jax.lax.broadcasted_iota(jnp.int32, (BC, BC), 0)
    c = jax.lax.broadcasted_iota(jnp.int32, (BC, BC), 1)
    eye_inf = jnp.where(r == c, jnp.inf, 0.0).astype(jnp.float32)

    for j in range(NB):
        cl, chi = j * BC, (j + 1) * BC
        rhs = wsbf[cl:chi, :]
        colmin = None
        base = 0
        while base < chi:
            sz = min(RC, chi - base)
            g = jax.lax.dot_general(
                wsbf[base:base + sz, :], rhs,
                dimension_numbers=(((1,), (1,)), ((), ())),
                preferred_element_type=jnp.float32,
            )
            if base <= cl < base + sz:
                loc = cl - base
                pieces = []
                if loc > 0:
                    pieces.append(g[:loc, :])
                pieces.append(g[loc:loc + BC, :] - eye_inf)
                if loc + BC < sz:
                    pieces.append(g[loc + BC:, :])
                g = (jnp.concatenate(pieces, axis=0)
                     if len(pieces) > 1 else pieces[0])
            t1 = sq_col[base:base + sz, :] - g
            part_col = jnp.min(t1, axis=0, keepdims=True)
            colmin = (part_col if colmin is None
                      else jnp.minimum(colmin, part_col))
            t2 = sq_row[:, cl:chi] - g
            part_row = jnp.minimum(
                jnp.minimum(t2[:, 0:128], t2[:, 128:256]),
                jnp.minimum(t2[:, 256:384], t2[:, 384:512]))
            rowacc[base:base + sz, :] = jnp.minimum(
                rowacc[base:base + sz, :], part_row)
            base += sz
        colacc[:, cl:chi] = colmin

    rowmin = jnp.min(rowacc[...], axis=1, keepdims=True)
    m = jnp.minimum(rowmin, jnp.reshape(colacc[...], (N, 1)))
    min_d2 = sq_col + m
    d = jnp.sqrt(jnp.maximum(min_d2, 1e-12))
    mean = jnp.sum(d) / N
    kept = jnp.where(d > mean, 0.0, d)
    loss = -(jnp.sum(kept) / N)
    out_ref[0] = loss
    out_ref[1] = mean


def kernel(weight):
    out = pl.pallas_call(
        _emb_loss_kernel,
        in_specs=[pl.BlockSpec((N, D), lambda: (0, 0))],
        out_specs=pl.BlockSpec(memory_space=pltpu.SMEM),
        out_shape=jax.ShapeDtypeStruct((2,), jnp.float32),
        scratch_shapes=[
            pltpu.VMEM((N, 128), jnp.float32),
            pltpu.VMEM((1, N), jnp.float32),
        ],
    )(weight)
    return out


# speedup vs baseline: 2.6399x; 1.0074x over previous
import jax
import jax.numpy as jnp
from jax.experimental import pallas as pl
from jax.experimental.pallas import tpu as pltpu

N = 8192
D = 512
BC = 512
NB = N // BC
RC = 4096
_SQRT2 = 1.4142135623730951


def _emb_loss_kernel(w_ref, out_ref, rowacc, colacc):
    w_full = w_ref[...]
    w2 = w_full * w_full

    sq_row = jax.lax.dot_general(
        jnp.ones((1, D), jnp.float32), w2,
        dimension_numbers=(((1,), (1,)), ((), ())),
        preferred_element_type=jnp.float32,
    )
    sq_col = jax.lax.dot_general(
        w2, jnp.ones((D, 1), jnp.float32),
        dimension_numbers=(((1,), (0,)), ((), ())),
        preferred_element_type=jnp.float32,
    )

    wsbf = (w_full * _SQRT2).astype(jnp.bfloat16)

    rowacc[...] = jnp.full((N, 128), jnp.inf, jnp.float32)

    r = jax.lax.broadcasted_iota(jnp.int32, (BC, BC), 0)
    c = jax.lax.broadcasted_iota(jnp.int32, (BC, BC), 1)
    eye_inf = jnp.where(r == c, jnp.inf, 0.0).astype(jnp.float32)

    for j in range(NB):
        cl, chi = j * BC, (j + 1) * BC
        rhs = wsbf[cl:chi, :]
        colmin = None
        base = 0
        while base < chi:
            sz = min(RC, chi - base)
            g = jax.lax.dot_general(
                wsbf[base:base + sz, :], rhs,
                dimension_numbers=(((1,), (1,)), ((), ())),
                preferred_element_type=jnp.float32,
            )
            if base <= cl < base + sz:
                loc = cl - base
                pieces = []
                if loc > 0:
                    pieces.append(g[:loc, :])
                pieces.append(g[loc:loc + BC, :] - eye_inf)
                if loc + BC < sz:
                    pieces.append(g[loc + BC:, :])
                g = (jnp.concatenate(pieces, axis=0)
                     if len(pieces) > 1 else pieces[0])
            t1 = sq_col[base:base + sz, :] - g
            part_col = jnp.min(t1, axis=0, keepdims=True)
            colmin = (part_col if colmin is None
                      else jnp.minimum(colmin, part_col))
            t2 = sq_row[:, cl:chi] - g
            part_row = jnp.minimum(
                jnp.minimum(t2[:, 0:128], t2[:, 128:256]),
                jnp.minimum(t2[:, 256:384], t2[:, 384:512]))
            rowacc[base:base + sz, :] = jnp.minimum(
                rowacc[base:base + sz, :], part_row)
            base += sz
        colacc[:, cl:chi] = colmin

    rowmin = jnp.min(rowacc[...], axis=1, keepdims=True)
    m = jnp.minimum(rowmin, jnp.reshape(colacc[...], (N, 1)))
    min_d2 = sq_col + m
    d = jnp.sqrt(jnp.maximum(min_d2, 1e-12))
    mean = jnp.sum(d) / N
    kept = jnp.where(d > mean, 0.0, d)
    loss = -(jnp.sum(kept) / N)
    out_ref[0] = loss
    out_ref[1] = mean


def kernel(weight):
    out = pl.pallas_call(
        _emb_loss_kernel,
        in_specs=[pl.BlockSpec((N, D), lambda: (0, 0))],
        out_specs=pl.BlockSpec(memory_space=pltpu.SMEM),
        out_shape=jax.ShapeDtypeStruct((2,), jnp.float32),
        scratch_shapes=[
            pltpu.VMEM((N, 128), jnp.float32),
            pltpu.VMEM((1, N), jnp.float32),
        ],
    )(weight)
    return out


# speedup vs baseline: 2.7186x; 1.0298x over previous
import jax
import jax.numpy as jnp
from jax.experimental import pallas as pl
from jax.experimental.pallas import tpu as pltpu

N = 8192
D = 512
BC = 512
NB = N // BC
RC = 2048
_SQRT2 = 1.4142135623730951


def _emb_loss_kernel(w_ref, out_ref, rowacc, colacc):
    w_full = w_ref[...]
    w2 = w_full * w_full

    sq_row = jax.lax.dot_general(
        jnp.ones((1, D), jnp.float32), w2,
        dimension_numbers=(((1,), (1,)), ((), ())),
        preferred_element_type=jnp.float32,
    )
    sq_col = jax.lax.dot_general(
        w2, jnp.ones((D, 1), jnp.float32),
        dimension_numbers=(((1,), (0,)), ((), ())),
        preferred_element_type=jnp.float32,
    )

    wsbf = (w_full * _SQRT2).astype(jnp.bfloat16)

    sq_row_bf = sq_row.astype(jnp.bfloat16)
    sq_col_bf = sq_col.astype(jnp.bfloat16)

    rowacc[...] = jnp.full((N, 128), jnp.inf, jnp.float32)

    r = jax.lax.broadcasted_iota(jnp.int32, (BC, BC), 0)
    c = jax.lax.broadcasted_iota(jnp.int32, (BC, BC), 1)
    eye_inf = jnp.where(r == c, jnp.inf, 0.0).astype(jnp.bfloat16)

    for j in range(NB):
        cl, chi = j * BC, (j + 1) * BC
        rhs = wsbf[cl:chi, :]
        colmin = None
        base = 0
        while base < chi:
            sz = min(RC, chi - base)
            g = jax.lax.dot_general(
                wsbf[base:base + sz, :], rhs,
                dimension_numbers=(((1,), (1,)), ((), ())),
                preferred_element_type=jnp.float32,
            ).astype(jnp.bfloat16)
            if base <= cl < base + sz:
                loc = cl - base
                pieces = []
                if loc > 0:
                    pieces.append(g[:loc, :])
                pieces.append(g[loc:loc + BC, :] - eye_inf)
                if loc + BC < sz:
                    pieces.append(g[loc + BC:, :])
                g = (jnp.concatenate(pieces, axis=0)
                     if len(pieces) > 1 else pieces[0])
            t1 = sq_col_bf[base:base + sz, :] - g
            part_col = jnp.min(t1, axis=0, keepdims=True)
            colmin = (part_col if colmin is None
                      else jnp.minimum(colmin, part_col))
            t2 = sq_row_bf[:, cl:chi] - g
            part_row = jnp.minimum(
                jnp.minimum(t2[:, 0:128], t2[:, 128:256]),
                jnp.minimum(t2[:, 256:384], t2[:, 384:512]))
            rowacc[base:base + sz, :] = jnp.minimum(
                rowacc[base:base + sz, :], part_row.astype(jnp.float32))
            base += sz
        colacc[:, cl:chi] = colmin.astype(jnp.float32)

    rowmin = jnp.min(rowacc[...], axis=1, keepdims=True)
    m = jnp.minimum(rowmin, jnp.reshape(colacc[...], (N, 1)))
    min_d2 = sq_col + m
    d = jnp.sqrt(jnp.maximum(min_d2, 1e-12))
    mean = jnp.sum(d) / N
    kept = jnp.where(d > mean, 0.0, d)
    loss = -(jnp.sum(kept) / N)
    out_ref[0] = loss
    out_ref[1] = mean


def kernel(weight):
    out = pl.pallas_call(
        _emb_loss_kernel,
        in_specs=[pl.BlockSpec((N, D), lambda: (0, 0))],
        out_specs=pl.BlockSpec(memory_space=pltpu.SMEM),
        out_shape=jax.ShapeDtypeStruct((2,), jnp.float32),
        scratch_shapes=[
            pltpu.VMEM((N, 128), jnp.float32),
            pltpu.VMEM((1, N), jnp.float32),
        ],
    )(weight)
    return out


# speedup vs baseline: 2.7189x; 1.0001x over previous
import jax
import jax.numpy as jnp
from jax.experimental import pallas as pl
from jax.experimental.pallas import tpu as pltpu

N = 8192
D = 512
BC = 512
NB = N // BC
RC = 2048
_SQRT2 = 1.4142135623730951


def _emb_loss_kernel(w_ref, out_ref, rowacc, colacc):
    w_full = w_ref[...]
    w2 = w_full * w_full

    sq_row = jax.lax.dot_general(
        jnp.ones((1, D), jnp.float32), w2,
        dimension_numbers=(((1,), (1,)), ((), ())),
        preferred_element_type=jnp.float32,
    )
    sq_col = jax.lax.dot_general(
        w2, jnp.ones((D, 1), jnp.float32),
        dimension_numbers=(((1,), (0,)), ((), ())),
        preferred_element_type=jnp.float32,
    )

    wsbf = (w_full * _SQRT2).astype(jnp.bfloat16)

    sq_row_bf = sq_row.astype(jnp.bfloat16)
    sq_col_bf = sq_col.astype(jnp.bfloat16)

    rowacc[...] = jnp.full((N, 128), jnp.inf, jnp.float32)

    r = jax.lax.broadcasted_iota(jnp.int32, (BC, BC), 0)
    c = jax.lax.broadcasted_iota(jnp.int32, (BC, BC), 1)
    eye_inf = jnp.where(r == c, jnp.inf, 0.0).astype(jnp.bfloat16)

    for j in range(NB):
        cl, chi = j * BC, (j + 1) * BC
        rhs = wsbf[cl:chi, :]
        sq_c_blk = sq_row_bf[:, cl:chi]
        colmin = None

        pieces = [(base, min(RC, cl - base), False)
                  for base in range(0, cl, RC)]
        pieces.append((cl, BC, True))

        for base, sz, is_diag in pieces:
            g = jax.lax.dot_general(
                wsbf[base:base + sz, :], rhs,
                dimension_numbers=(((1,), (1,)), ((), ())),
                preferred_element_type=jnp.float32,
            ).astype(jnp.bfloat16)
            if is_diag:
                g = g - eye_inf
            t1 = sq_col_bf[base:base + sz, :] - g
            part_col = jnp.min(t1, axis=0, keepdims=True)
            colmin = (part_col if colmin is None
                      else jnp.minimum(colmin, part_col))
            t2 = sq_c_blk - g
            part_row = jnp.minimum(
                jnp.minimum(t2[:, 0:128], t2[:, 128:256]),
                jnp.minimum(t2[:, 256:384], t2[:, 384:512]))
            rowacc[base:base + sz, :] = jnp.minimum(
                rowacc[base:base + sz, :], part_row.astype(jnp.float32))
        colacc[:, cl:chi] = colmin.astype(jnp.float32)

    rowmin = jnp.min(rowacc[...], axis=1, keepdims=True)
    m = jnp.minimum(rowmin, jnp.reshape(colacc[...], (N, 1)))
    min_d2 = sq_col + m
    d = jnp.sqrt(jnp.maximum(min_d2, 1e-12))
    mean = jnp.sum(d) / N
    kept = jnp.where(d > mean, 0.0, d)
    loss = -(jnp.sum(kept) / N)
    out_ref[0] = loss
    out_ref[1] = mean


def kernel(weight):
    out = pl.pallas_call(
        _emb_loss_kernel,
        in_specs=[pl.BlockSpec((N, D), lambda: (0, 0))],
        out_specs=pl.BlockSpec(memory_space=pltpu.SMEM),
        out_shape=jax.ShapeDtypeStruct((2,), jnp.float32),
        scratch_shapes=[
            pltpu.VMEM((N, 128), jnp.float32),
            pltpu.VMEM((1, N), jnp.float32),
        ],
    )(weight)
    return out
